# Initial kernel scaffold; baseline (speedup 1.0000x reference)
import jax
import jax.numpy as jnp
from jax.experimental import pallas as pl


def kernel(species, positions):
    raise NotImplementedError("write your pallas kernel here")


# speedup vs baseline: 16.6365x; 16.6365x over previous
import numpy as np
import jax
import jax.numpy as jnp
from jax.experimental import pallas as pl

_NSP = 4
_RCR = 5.2
_RCA = 3.5
_ETAR = 16.0
_ETAA = 8.0
_SHFR = (0.9 + 0.26875 * np.arange(16)).astype(np.float32)
_SHFA = np.array([0.9, 1.55, 2.2, 2.85], dtype=np.float32)
_SHFZ = ((np.arange(8) + 0.5) * np.pi / 8.0).astype(np.float32)
_COSZ = np.cos(_SHFZ)
_SINZ = np.sin(_SHFZ)
_K = 24
_NQ = 10

_JJ, _KK = np.triu_indices(_K, 1)
_P = _JJ.size
_SELJ = np.zeros((_K, _P), dtype=np.float32)
_SELJ[_JJ, np.arange(_P)] = 1.0
_SELK = np.zeros((_K, _P), dtype=np.float32)
_SELK[_KK, np.arange(_P)] = 1.0

_BI = 128


def _cut(d, rc):
    return jnp.where(d < rc, 0.5 * jnp.cos(np.float32(np.pi / rc) * jnp.minimum(d, rc)) + 0.5, 0.0)


def _pow32(x):
    x = x * x
    x = x * x
    x = x * x
    x = x * x
    return x * x


def _ani_kernel(posi_ref, posT_ref, spcol_ref, sprow_ref, selj_ref, selk_ref, out_ref):
    i = pl.program_id(0)
    bi = posi_ref.shape[0]
    n = posT_ref.shape[1]

    xi = posi_ref[:, 0:1]
    yi = posi_ref[:, 1:2]
    zi = posi_ref[:, 2:3]
    xr = posT_ref[0:1, :]
    yr = posT_ref[1:2, :]
    zr = posT_ref[2:3, :]

    dxm = xi - xr
    dym = yi - yr
    dzm = zi - zr
    dist = jnp.sqrt(dxm * dxm + dym * dym + dzm * dzm + 1e-12)

    jio_i = jax.lax.broadcasted_iota(jnp.int32, (bi, n), 1)
    row_i = jax.lax.broadcasted_iota(jnp.int32, (bi, n), 0) + i * bi
    dist = jnp.where(jio_i == row_i, 1e6, dist)

    fc_r = _cut(dist, _RCR)
    oh = (spcol_ref[:, 0:1] == jax.lax.broadcasted_iota(jnp.int32, (n, _NSP), 1)).astype(jnp.float32)
    rads = []
    for m in range(16):
        dd = dist - _SHFR[m]
        t = (0.25 * jnp.exp(-_ETAR * dd * dd)) * fc_r
        rads.append(jax.lax.dot_general(t, oh, (((1,), (0,)), ((), ())),
                                        preferred_element_type=jnp.float32))
    radial = jnp.stack(rads, axis=2).reshape(bi, _NSP * 16)

    jio_f = jio_i.astype(jnp.float32)
    sprow_f = sprow_ref[0:1, :].astype(jnp.float32)
    dwork = jnp.where(dist < _RCA, dist, 1e6)
    nds, ndx, ndy, ndz, nsp = [], [], [], [], []
    for _ in range(_K):
        mval = jnp.min(dwork, axis=1, keepdims=True)
        aidx = jnp.min(jnp.where(dwork == mval, jio_f, 1e9), axis=1, keepdims=True)
        sel = jio_f == aidx
        nds.append(mval)
        ndx.append(jnp.sum(jnp.where(sel, xr, 0.0), axis=1, keepdims=True) - xi)
        ndy.append(jnp.sum(jnp.where(sel, yr, 0.0), axis=1, keepdims=True) - yi)
        ndz.append(jnp.sum(jnp.where(sel, zr, 0.0), axis=1, keepdims=True) - zi)
        nsp.append(jnp.sum(jnp.where(sel, sprow_f, 0.0), axis=1, keepdims=True))
        dwork = jnp.where(sel, 1e6, dwork)
    nd = jnp.concatenate(nds, axis=1)
    nx = jnp.concatenate(ndx, axis=1)
    ny = jnp.concatenate(ndy, axis=1)
    nz = jnp.concatenate(ndz, axis=1)
    ns = jnp.concatenate(nsp, axis=1)

    selj = selj_ref[...]
    selk = selk_ref[...]

    def pick(v, s):
        return jax.lax.dot_general(v, s, (((1,), (0,)), ((), ())),
                                   preferred_element_type=jnp.float32)

    d1 = pick(nd, selj)
    d2 = pick(nd, selk)
    x1 = pick(nx, selj)
    x2 = pick(nx, selk)
    y1 = pick(ny, selj)
    y2 = pick(ny, selk)
    z1 = pick(nz, selj)
    z2 = pick(nz, selk)
    s1 = pick(ns, selj)
    s2 = pick(ns, selk)
    vf = (nd < _RCA).astype(jnp.float32)
    pv = pick(vf, selj) * pick(vf, selk)

    dotp = x1 * x2 + y1 * y2 + z1 * z2
    c = 0.95 * dotp / (d1 * d2 + 1e-12)
    c = jnp.clip(c, -0.999999, 0.999999)
    s = jnp.sqrt(1.0 - c * c)
    base = 2.0 * _cut(d1, _RCA) * _cut(d2, _RCA) * pv

    davg = 0.5 * (d1 + d2)
    f2 = []
    for q in range(4):
        dd = davg - _SHFA[q]
        f2.append(jnp.exp(-_ETAA * dd * dd))
    f1 = []
    for z in range(8):
        u = 0.5 * (1.0 + c * _COSZ[z] + s * _SINZ[z])
        f1.append(_pow32(u))

    a = jnp.minimum(s1, s2)
    b = jnp.maximum(s1, s2)
    pidx = a * 4.0 - a * (a - 1.0) * 0.5 + (b - a)

    cols = []
    for q in range(_NQ):
        wq = jnp.where(pidx == np.float32(q), base, 0.0)
        for z in range(8):
            az = wq * f1[z]
            for t in range(4):
                cols.append(jnp.sum(az * f2[t], axis=1, keepdims=True))
    angular = jnp.concatenate(cols, axis=1)

    out_ref[0] = jnp.concatenate([radial, angular], axis=1)


def kernel(species, positions):
    sp = species[0]
    pos = positions[0]
    n = pos.shape[0]
    posT = pos.T
    spcol = sp[:, None]
    sprow = sp[None, :]
    selj = jnp.asarray(_SELJ)
    selk = jnp.asarray(_SELK)

    feats = pl.pallas_call(
        _ani_kernel,
        grid=(n // _BI,),
        in_specs=[
            pl.BlockSpec((_BI, 3), lambda i: (i, 0)),
            pl.BlockSpec((3, n), lambda i: (0, 0)),
            pl.BlockSpec((n, 1), lambda i: (0, 0)),
            pl.BlockSpec((1, n), lambda i: (0, 0)),
            pl.BlockSpec((_K, _P), lambda i: (0, 0)),
            pl.BlockSpec((_K, _P), lambda i: (0, 0)),
        ],
        out_specs=pl.BlockSpec((1, _BI, 384), lambda i: (0, i, 0)),
        out_shape=jax.ShapeDtypeStruct((1, n, 384), jnp.float32),
    )(pos, posT, spcol, sprow, selj, selk)

    return species, feats
